# baseline (speedup 1.0000x reference)
import functools

import jax
import jax.numpy as jnp
from jax import lax
from jax.experimental import pallas as pl
from jax.experimental.pallas import tpu as pltpu
from jax.experimental.pallas import tpu_sc as plsc

VOCAB = 1000000
EMBED_DIM = 64
BATCH = 4096
SEQ = 20

_INFO = plsc.get_sparse_core_info()
_NC, _NS = _INFO.num_cores, _INFO.num_subcores
_NW = _NC * _NS
_TOTAL = BATCH * SEQ
_CHUNK = 128
_PER_W = _TOTAL // _NW
_NCHUNK = _PER_W // _CHUNK


def _gather_body(table_hbm, idx_hbm, out_hbm, idx_v, rows_v, sem):
    wid = lax.axis_index("s") * _NC + lax.axis_index("c")
    base = wid * _PER_W
    pltpu.sync_copy(idx_hbm.at[wid], idx_v)

    def chunk(j, carry):
        pltpu.async_copy(table_hbm.at[idx_v.at[j]], rows_v, sem).wait()
        pltpu.sync_copy(rows_v, out_hbm.at[pl.ds(base + j * _CHUNK, _CHUNK)])
        return carry

    lax.fori_loop(0, _NCHUNK, chunk, 0)


@jax.jit
def _embed_gather(x_flat, table):
    mesh = plsc.VectorSubcoreMesh(core_axis_name="c", subcore_axis_name="s")
    k = pl.kernel(
        _gather_body,
        out_type=jax.ShapeDtypeStruct((_TOTAL, EMBED_DIM), jnp.float32),
        mesh=mesh,
        scratch_types=[
            pltpu.VMEM((_NCHUNK, _CHUNK), jnp.int32),
            pltpu.VMEM((_CHUNK, EMBED_DIM), jnp.float32),
            pltpu.SemaphoreType.DMA,
        ],
        compiler_params=pltpu.CompilerParams(use_tc_tiling_on_sc=False),
    )
    return k(table, x_flat.reshape(_NW, _NCHUNK, _CHUNK))


def kernel(x, input_embedding):
    out = _embed_gather(x.reshape(-1), input_embedding)
    return out.reshape(BATCH, SEQ, EMBED_DIM)


# speedup vs baseline: 1.0176x; 1.0176x over previous
import functools

import jax
import jax.numpy as jnp
from jax import lax
from jax.experimental import pallas as pl
from jax.experimental.pallas import tpu as pltpu
from jax.experimental.pallas import tpu_sc as plsc

VOCAB = 1000000
EMBED_DIM = 64
BATCH = 4096
SEQ = 20

_INFO = plsc.get_sparse_core_info()
_NC, _NS = _INFO.num_cores, _INFO.num_subcores
_NW = _NC * _NS
_TOTAL = BATCH * SEQ
_CHUNK = 128
_PER_W = _TOTAL // _NW
_NCHUNK = _PER_W // _CHUNK


_NBUF = 8
_LAG = 4


def _gather_body(table_hbm, idx_hbm, out_hbm, idx_v, rows_v, gsem, wsem):
    wid = lax.axis_index("s") * _NC + lax.axis_index("c")
    base = wid * _PER_W
    pltpu.sync_copy(idx_hbm.at[wid], idx_v)

    gathers = [None] * _NCHUNK
    writes = [None] * _NCHUNK
    for j in range(_NCHUNK):
        b = j % _NBUF
        if j >= _NBUF:
            writes[j - _NBUF].wait()
        gathers[j] = pltpu.async_copy(
            table_hbm.at[idx_v.at[j]], rows_v.at[b], gsem.at[b]
        )
        if j >= _LAG:
            i = j - _LAG
            gathers[i].wait()
            writes[i] = pltpu.async_copy(
                rows_v.at[i % _NBUF],
                out_hbm.at[pl.ds(base + i * _CHUNK, _CHUNK)],
                wsem.at[i % _NBUF],
            )
    for i in range(_NCHUNK - _LAG, _NCHUNK):
        gathers[i].wait()
        writes[i] = pltpu.async_copy(
            rows_v.at[i % _NBUF],
            out_hbm.at[pl.ds(base + i * _CHUNK, _CHUNK)],
            wsem.at[i % _NBUF],
        )
    for i in range(_NCHUNK - _NBUF, _NCHUNK):
        writes[i].wait()


@jax.jit
def _embed_gather(x_flat, table):
    mesh = plsc.VectorSubcoreMesh(core_axis_name="c", subcore_axis_name="s")
    k = pl.kernel(
        _gather_body,
        out_type=jax.ShapeDtypeStruct((_TOTAL, EMBED_DIM), jnp.float32),
        mesh=mesh,
        scratch_types=[
            pltpu.VMEM((_NCHUNK, _CHUNK), jnp.int32),
            pltpu.VMEM((_NBUF, _CHUNK, EMBED_DIM), jnp.float32),
            pltpu.SemaphoreType.DMA((_NBUF,)),
            pltpu.SemaphoreType.DMA((_NBUF,)),
        ],
        compiler_params=pltpu.CompilerParams(use_tc_tiling_on_sc=False),
    )
    return k(table, x_flat.reshape(_NW, _NCHUNK, _CHUNK))


def kernel(x, input_embedding):
    out = _embed_gather(x.reshape(-1), input_embedding)
    return out.reshape(BATCH, SEQ, EMBED_DIM)
